# baseline (speedup 1.0000x reference)
import functools

import jax
import jax.numpy as jnp
from jax import lax
from jax.experimental import pallas as pl
from jax.experimental.pallas import tpu as pltpu
from jax.experimental.pallas import tpu_sc as plsc

N_NODES = 10000
N_MESH = 320000
N_WORLD = 32000
D = 128

NC = 2
NS = 16
NW = NC * NS

MB = 80
M_PER_TILE = N_MESH // NW
M_CHUNKS = M_PER_TILE // MB
WB = 40
W_PER_TILE = N_WORLD // NW
W_CHUNKS = W_PER_TILE // WB

GM = 80
GM_OUT = M_PER_TILE // GM
M_SUBS = GM // MB
GW = 40
GW_OUT = W_PER_TILE // GW
W_SUBS = GW // WB
NB = 4
ROWS_A = 632
ROWS_B = N_NODES - 15 * ROWS_A


def _sc_aggregate(edge_attr, mesh_idx, world_attr, world_idx, zeros):
  mesh = plsc.VectorSubcoreMesh(core_axis_name="c", subcore_axis_name="s",
                                num_cores=NC, num_subcores=NS)

  @functools.partial(
      pl.kernel,
      out_type=[
          jax.ShapeDtypeStruct((NC, N_NODES, D), jnp.float32),
          jax.ShapeDtypeStruct((NC, N_NODES, D), jnp.float32),
      ],
      mesh=mesh,
      scratch_types=[
          *[pltpu.VMEM((GM, D), jnp.float32) for _ in range(NB)],
          *[pltpu.VMEM((MB,), jnp.int32) for _ in range(NB)],
          *[pltpu.VMEM((WB,), jnp.int32) for _ in range(NB)],
          pltpu.VMEM_SHARED((N_NODES, D), jnp.float32),
          *[pltpu.SemaphoreType.DMA for _ in range(2 * NB)],
      ],
  )
  def k(edge_hbm, midx_hbm, world_hbm, widx_hbm, zeros_hbm,
        mesh_out, world_out, *scratch):
    ebufs = scratch[:NB]
    ibufs = scratch[NB:2 * NB]
    wibufs = scratch[2 * NB:3 * NB]
    acc = scratch[3 * NB]
    fsems = scratch[3 * NB + 1:3 * NB + 1 + NB]
    ssems = scratch[3 * NB + 1 + NB:]
    c = lax.axis_index("c")
    s = lax.axis_index("s")
    t = c * NS + s
    r0 = s * ROWS_A

    def acc_edge_copy(src_ref, dst_ref):
      @pl.when(s < NS - 1)
      def _():
        pltpu.sync_copy(src_ref.at[pl.ds(r0, ROWS_A)],
                        dst_ref.at[pl.ds(r0, ROWS_A)])

      @pl.when(s == NS - 1)
      def _():
        pltpu.sync_copy(src_ref.at[pl.ds(15 * ROWS_A, ROWS_B)],
                        dst_ref.at[pl.ds(15 * ROWS_A, ROWS_B)])

    def run_phase(nout, src, isrc, rows):
      def buf(b):
        return ebufs[b] if rows == GM else ebufs[b].at[pl.ds(0, rows)]

      def ibuf(b):
        return ibufs[b] if rows == GM else wibufs[b]

      def fire_fetch(g, b):
        pltpu.async_copy(src(g), buf(b), fsems[b])
        pltpu.async_copy(isrc(g), ibuf(b), fsems[b])

      def slot(u, bu, first=False, fire=True):
        pltpu.make_async_copy(src(u), buf(bu), fsems[bu]).wait()
        pltpu.make_async_copy(isrc(u), ibuf(bu), fsems[bu]).wait()
        pltpu.async_copy(buf(bu), acc.at[ibuf(bu)], ssems[bu], add=True)
        bp = (bu + 2) % NB
        if not first:
          pltpu.make_async_copy(buf(bp), acc.at[ibuf(bp)], ssems[bp]).wait()
        if fire:
          fire_fetch(u + 2, bp)

      fire_fetch(0, 0)
      fire_fetch(1, 1)
      for u in range(NB - 2):
        slot(u, u, first=True)

      m0 = NB - 2
      n_main = ((nout - m0 - 3) // NB) * NB

      @pl.loop(m0, m0 + n_main, step=NB)
      def _(g):
        for d in range(NB):
          slot(g + d, (m0 + d) % NB)

      for u in range(m0 + n_main, nout):
        slot(u, u % NB, fire=(u + 2 < nout))
      for u in range(nout - (NB - 2), nout):
        pltpu.make_async_copy(buf(u % NB), acc.at[ibuf(u % NB)],
                              ssems[u % NB]).wait()

    acc_edge_copy(zeros_hbm, acc)
    plsc.subcore_barrier()

    mbase = t * M_PER_TILE
    run_phase(GM_OUT, lambda g: edge_hbm.at[pl.ds(mbase + g * GM, GM)],
              lambda g: midx_hbm.at[pl.ds(mbase + g * GM, GM)], GM)
    plsc.subcore_barrier()
    acc_edge_copy(acc, mesh_out.at[c])
    plsc.subcore_barrier()

    acc_edge_copy(zeros_hbm, acc)
    plsc.subcore_barrier()

    wbase = t * W_PER_TILE
    run_phase(GW_OUT, lambda g: world_hbm.at[pl.ds(wbase + g * GW, GW)],
              lambda g: widx_hbm.at[pl.ds(wbase + g * GW, GW)], GW)
    plsc.subcore_barrier()
    acc_edge_copy(acc, world_out.at[c])

  return k(edge_attr, mesh_idx, world_attr, world_idx, zeros)


ROWS_BLK = 1000


def _tc_mlp_body(x, mp0, mp1, wp0, wp1, w1a, w1b, w1c, b1, w2, b2, out):
  m = mp0[0] + mp1[0]
  w = wp0[0] + wp1[0]
  h = (jnp.dot(x[...], w1a[...], preferred_element_type=jnp.float32)
       + jnp.dot(m, w1b[...], preferred_element_type=jnp.float32)
       + jnp.dot(w, w1c[...], preferred_element_type=jnp.float32)
       + b1[...])
  h = jnp.maximum(h, 0.0)
  out[...] = (jnp.dot(h, w2[...], preferred_element_type=jnp.float32)
              + b2[...])


def _tc_mlp(x, mesh_parts, world_parts, W1, b1, W2, b2):
  w1a, w1b, w1c = W1[:D], W1[D:2 * D], W1[2 * D:]
  b1r = b1.reshape(1, D)
  b2r = b2.reshape(1, D)
  rows_spec = pl.BlockSpec((ROWS_BLK, D), lambda i: (i, 0))
  part0_spec = pl.BlockSpec((1, ROWS_BLK, D), lambda i: (0, i, 0))
  part1_spec = pl.BlockSpec((1, ROWS_BLK, D), lambda i: (1, i, 0))
  full_spec = pl.BlockSpec((D, D), lambda i: (0, 0))
  bias_spec = pl.BlockSpec((1, D), lambda i: (0, 0))
  return pl.pallas_call(
      _tc_mlp_body,
      grid=(N_NODES // ROWS_BLK,),
      in_specs=[rows_spec, part0_spec, part1_spec, part0_spec, part1_spec,
                full_spec, full_spec, full_spec, bias_spec, full_spec,
                bias_spec],
      out_specs=rows_spec,
      out_shape=jax.ShapeDtypeStruct((N_NODES, D), jnp.float32),
  )(x, mesh_parts, mesh_parts, world_parts, world_parts,
    w1a, w1b, w1c, b1r, W2, b2r)


def kernel(x, edge_attr, edge_index, world_edge_attr, world_edge_index,
           W1, b1, W2, b2):
  mesh_idx = edge_index[1].astype(jnp.int32)
  world_idx = world_edge_index[1].astype(jnp.int32)
  zeros = jnp.zeros((N_NODES, D), jnp.float32)
  mesh_parts, world_parts = _sc_aggregate(
      edge_attr, mesh_idx, world_edge_attr, world_idx, zeros)
  return _tc_mlp(x, mesh_parts, world_parts, W1, b1, W2, b2)


# speedup vs baseline: 1.1356x; 1.1356x over previous
import functools

import jax
import jax.numpy as jnp
from jax import lax
from jax.experimental import pallas as pl
from jax.experimental.pallas import tpu as pltpu
from jax.experimental.pallas import tpu_sc as plsc

N_NODES = 10000
N_MESH = 320000
N_WORLD = 32000
D = 128

NC = 2
NS = 16
NW = NC * NS

MB = 80
M_PER_TILE = N_MESH // NW
M_CHUNKS = M_PER_TILE // MB
WB = 40
W_PER_TILE = N_WORLD // NW
W_CHUNKS = W_PER_TILE // WB

GM = 80
GM_OUT = M_PER_TILE // GM
M_SUBS = GM // MB
GW = 40
GW_OUT = W_PER_TILE // GW
W_SUBS = GW // WB
N_PAD = 10240
ROWS_PER_TILE = N_PAD // NS


def _sc_aggregate(edge_attr, mesh_idx, world_attr, world_idx, zeros):
  mesh = plsc.VectorSubcoreMesh(core_axis_name="c", subcore_axis_name="s",
                                num_cores=NC, num_subcores=NS)

  @functools.partial(
      pl.kernel,
      out_type=[
          jax.ShapeDtypeStruct((NC, N_PAD, D), jnp.float32),
          jax.ShapeDtypeStruct((NC, N_PAD, D), jnp.float32),
      ],
      mesh=mesh,
      scratch_types=[
          pltpu.VMEM((GM, D), jnp.float32),
          pltpu.VMEM((GM, D), jnp.float32),
          pltpu.VMEM((GM, D), jnp.float32),
          pltpu.VMEM((MB,), jnp.int32),
          pltpu.VMEM((MB,), jnp.int32),
          pltpu.VMEM((MB,), jnp.int32),
          pltpu.VMEM((WB,), jnp.int32),
          pltpu.VMEM((WB,), jnp.int32),
          pltpu.VMEM((WB,), jnp.int32),
          pltpu.VMEM_SHARED((N_PAD, D), jnp.float32),
          pltpu.SemaphoreType.DMA,
          pltpu.SemaphoreType.DMA,
          pltpu.SemaphoreType.DMA,
          pltpu.SemaphoreType.DMA,
          pltpu.SemaphoreType.DMA,
          pltpu.SemaphoreType.DMA,
      ],
  )
  def k(edge_hbm, midx_hbm, world_hbm, widx_hbm, zeros_hbm,
        mesh_out, world_out, ebuf0, ebuf1, ebuf2, ibuf0, ibuf1, ibuf2,
        wibuf0, wibuf1, wibuf2, acc,
        fsem0, fsem1, fsem2, ssem0, ssem1, ssem2):
    c = lax.axis_index("c")
    s = lax.axis_index("s")
    t = c * NS + s
    r0 = s * ROWS_PER_TILE
    ebufs = (ebuf0, ebuf1, ebuf2)
    ibufs = (ibuf0, ibuf1, ibuf2)
    wibufs = (wibuf0, wibuf1, wibuf2)
    fsems = (fsem0, fsem1, fsem2)
    ssems = (ssem0, ssem1, ssem2)

    def run_phase(nout, src, isrc, rows, primed=False):
      def buf(b):
        return ebufs[b] if rows == GM else ebufs[b].at[pl.ds(0, rows)]

      def ibuf(b):
        return ibufs[b] if rows == GM else wibufs[b]

      def fire_fetch(g, b):
        pltpu.async_copy(src(g), buf(b), fsems[b])
        pltpu.async_copy(isrc(g), ibuf(b), fsems[b])

      def slot(u, bu, first=False, fire=True):
        pltpu.make_async_copy(src(u), buf(bu), fsems[bu]).wait()
        pltpu.make_async_copy(isrc(u), ibuf(bu), fsems[bu]).wait()
        pltpu.async_copy(buf(bu), acc.at[ibuf(bu)], ssems[bu], add=True)
        bp = (bu + 2) % 3
        if not first:
          pltpu.make_async_copy(buf(bp), acc.at[ibuf(bp)], ssems[bp]).wait()
        if fire:
          fire_fetch(u + 2, bp)

      if not primed:
        fire_fetch(0, 0)
        fire_fetch(1, 1)
      slot(0, 0, first=True)

      n_main = ((nout - 4 - 1) // 3) * 3

      @pl.loop(1, 1 + n_main, step=3)
      def _(g):
        slot(g, 1 % 3)
        slot(g + 1, 2 % 3)
        slot(g + 2, 0)

      for u in range(1 + n_main, nout):
        slot(u, u % 3, fire=(u + 2 < nout))
      b_last = (nout - 1) % 3
      pltpu.make_async_copy(buf(b_last), acc.at[ibuf(b_last)],
                            ssems[b_last]).wait()

    def zero_acc_slice():
      pltpu.sync_copy(zeros_hbm, ebuf2)
      for z in range(ROWS_PER_TILE // GM):
        pltpu.sync_copy(ebuf2, acc.at[pl.ds(r0 + z * GM, GM)])

    mbase = t * M_PER_TILE
    msrc = lambda g: edge_hbm.at[pl.ds(mbase + g * GM, GM)]
    misrc = lambda g: midx_hbm.at[pl.ds(mbase + g * GM, GM)]
    pltpu.async_copy(msrc(0), ebuf0, fsem0)
    pltpu.async_copy(misrc(0), ibuf0, fsem0)
    pltpu.async_copy(msrc(1), ebuf1, fsem1)
    pltpu.async_copy(misrc(1), ibuf1, fsem1)
    zero_acc_slice()
    plsc.subcore_barrier()

    run_phase(GM_OUT, msrc, misrc, GM, primed=True)
    wbase = t * W_PER_TILE
    wsrc = lambda g: world_hbm.at[pl.ds(wbase + g * GW, GW)]
    wisrc = lambda g: widx_hbm.at[pl.ds(wbase + g * GW, GW)]
    pltpu.async_copy(wsrc(0), ebuf0.at[pl.ds(0, GW)], fsem0)
    pltpu.async_copy(wisrc(0), wibuf0, fsem0)
    pltpu.async_copy(wsrc(1), ebuf1.at[pl.ds(0, GW)], fsem1)
    pltpu.async_copy(wisrc(1), wibuf1, fsem1)

    plsc.subcore_barrier()
    pltpu.sync_copy(acc.at[pl.ds(r0, ROWS_PER_TILE)],
                    mesh_out.at[c, pl.ds(r0, ROWS_PER_TILE)])
    plsc.subcore_barrier()

    zero_acc_slice()
    plsc.subcore_barrier()

    run_phase(GW_OUT, wsrc, wisrc, GW, primed=True)
    plsc.subcore_barrier()
    pltpu.sync_copy(acc.at[pl.ds(r0, ROWS_PER_TILE)],
                    world_out.at[c, pl.ds(r0, ROWS_PER_TILE)])

  return k(edge_attr, mesh_idx, world_attr, world_idx, zeros)


ROWS_BLK = 1000


def _tc_mlp_body(x, mp0, mp1, wp0, wp1, w1a, w1b, w1c, b1, w2, b2, out):
  m = mp0[0] + mp1[0]
  w = wp0[0] + wp1[0]
  h = (jnp.dot(x[...], w1a[...], preferred_element_type=jnp.float32)
       + jnp.dot(m, w1b[...], preferred_element_type=jnp.float32)
       + jnp.dot(w, w1c[...], preferred_element_type=jnp.float32)
       + b1[...])
  h = jnp.maximum(h, 0.0)
  out[...] = (jnp.dot(h, w2[...], preferred_element_type=jnp.float32)
              + b2[...])


def _tc_mlp(x, mesh_parts, world_parts, W1, b1, W2, b2):
  w1a, w1b, w1c = W1[:D], W1[D:2 * D], W1[2 * D:]
  b1r = b1.reshape(1, D)
  b2r = b2.reshape(1, D)
  rows_spec = pl.BlockSpec((ROWS_BLK, D), lambda i: (i, 0))
  part0_spec = pl.BlockSpec((1, ROWS_BLK, D), lambda i: (0, i, 0))
  part1_spec = pl.BlockSpec((1, ROWS_BLK, D), lambda i: (1, i, 0))
  full_spec = pl.BlockSpec((D, D), lambda i: (0, 0))
  bias_spec = pl.BlockSpec((1, D), lambda i: (0, 0))
  return pl.pallas_call(
      _tc_mlp_body,
      grid=(N_NODES // ROWS_BLK,),
      in_specs=[rows_spec, part0_spec, part1_spec, part0_spec, part1_spec,
                full_spec, full_spec, full_spec, bias_spec, full_spec,
                bias_spec],
      out_specs=rows_spec,
      out_shape=jax.ShapeDtypeStruct((N_NODES, D), jnp.float32),
  )(x, mesh_parts, mesh_parts, world_parts, world_parts,
    w1a, w1b, w1c, b1r, W2, b2r)


def kernel(x, edge_attr, edge_index, world_edge_attr, world_edge_index,
           W1, b1, W2, b2):
  mesh_idx = edge_index[1].astype(jnp.int32)
  world_idx = world_edge_index[1].astype(jnp.int32)
  zeros = jnp.zeros((GM, D), jnp.float32)
  mesh_parts, world_parts = _sc_aggregate(
      edge_attr, mesh_idx, world_edge_attr, world_idx, zeros)
  return _tc_mlp(x, mesh_parts, world_parts, W1, b1, W2, b2)


# speedup vs baseline: 1.2120x; 1.0672x over previous
import functools

import jax
import jax.numpy as jnp
from jax import lax
from jax.experimental import pallas as pl
from jax.experimental.pallas import tpu as pltpu
from jax.experimental.pallas import tpu_sc as plsc

N_NODES = 10000
N_MESH = 320000
N_WORLD = 32000
D = 128

NC = 2
NS = 16
NW = NC * NS

MB = 80
M_PER_TILE = N_MESH // NW
M_CHUNKS = M_PER_TILE // MB
WB = 40
W_PER_TILE = N_WORLD // NW
W_CHUNKS = W_PER_TILE // WB

GM = 80
GM_OUT = M_PER_TILE // GM
M_SUBS = GM // MB
GW = 40
GW_OUT = W_PER_TILE // GW
W_SUBS = GW // WB
N_PAD = 10240
ROWS_PER_TILE = N_PAD // NS


def _sc_aggregate(edge_attr, mesh_idx, world_attr, world_idx, zeros):
  mesh = plsc.VectorSubcoreMesh(core_axis_name="c", subcore_axis_name="s",
                                num_cores=NC, num_subcores=NS)

  @functools.partial(
      pl.kernel,
      out_type=[
          jax.ShapeDtypeStruct((NC, N_PAD, D), jnp.float32),
          jax.ShapeDtypeStruct((NC, N_PAD, D), jnp.float32),
      ],
      mesh=mesh,
      scratch_types=[
          pltpu.VMEM((GM, D), jnp.float32),
          pltpu.VMEM((GM, D), jnp.float32),
          pltpu.VMEM((GM, D), jnp.float32),
          pltpu.VMEM((MB,), jnp.int32),
          pltpu.VMEM((MB,), jnp.int32),
          pltpu.VMEM((MB,), jnp.int32),
          pltpu.VMEM((WB,), jnp.int32),
          pltpu.VMEM((WB,), jnp.int32),
          pltpu.VMEM((WB,), jnp.int32),
          pltpu.VMEM_SHARED((N_PAD, D), jnp.float32),
          pltpu.SemaphoreType.DMA,
          pltpu.SemaphoreType.DMA,
          pltpu.SemaphoreType.DMA,
          pltpu.SemaphoreType.DMA,
          pltpu.SemaphoreType.DMA,
          pltpu.SemaphoreType.DMA,
      ],
  )
  def k(edge_hbm, midx_hbm, world_hbm, widx_hbm, zeros_hbm,
        mesh_out, world_out, ebuf0, ebuf1, ebuf2, ibuf0, ibuf1, ibuf2,
        wibuf0, wibuf1, wibuf2, acc,
        fsem0, fsem1, fsem2, ssem0, ssem1, ssem2):
    c = lax.axis_index("c")
    s = lax.axis_index("s")
    t = c * NS + s
    r0 = s * ROWS_PER_TILE
    ebufs = (ebuf0, ebuf1, ebuf2)
    ibufs = (ibuf0, ibuf1, ibuf2)
    wibufs = (wibuf0, wibuf1, wibuf2)
    fsems = (fsem0, fsem1, fsem2)
    ssems = (ssem0, ssem1, ssem2)

    def run_phase(nout, src, isrc, rows, primed=False):
      def buf(b):
        return ebufs[b] if rows == GM else ebufs[b].at[pl.ds(0, rows)]

      def ibuf(b):
        return ibufs[b] if rows == GM else wibufs[b]

      def fire_fetch(g, b):
        pltpu.async_copy(src(g), buf(b), fsems[b])
        pltpu.async_copy(isrc(g), ibuf(b), fsems[b])

      def slot(u, bu, first=False, fire=True):
        pltpu.make_async_copy(src(u), buf(bu), fsems[bu]).wait()
        pltpu.make_async_copy(isrc(u), ibuf(bu), fsems[bu]).wait()
        pltpu.async_copy(buf(bu), acc.at[ibuf(bu)], ssems[bu], add=True)
        bp = (bu + 2) % 3
        if not first:
          pltpu.make_async_copy(buf(bp), acc.at[ibuf(bp)], ssems[bp]).wait()
        if fire:
          fire_fetch(u + 2, bp)

      if not primed:
        fire_fetch(0, 0)
        fire_fetch(1, 1)
      slot(0, 0, first=True)

      n_main = ((nout - 4 - 1) // 3) * 3

      @pl.loop(1, 1 + n_main, step=3)
      def _(g):
        slot(g, 1 % 3)
        slot(g + 1, 2 % 3)
        slot(g + 2, 0)

      for u in range(1 + n_main, nout):
        slot(u, u % 3, fire=(u + 2 < nout))
      b_last = (nout - 1) % 3
      pltpu.make_async_copy(buf(b_last), acc.at[ibuf(b_last)],
                            ssems[b_last]).wait()

    def zero_acc_slice():
      pltpu.sync_copy(zeros_hbm, ebuf2)
      for z in range(ROWS_PER_TILE // GM):
        pltpu.sync_copy(ebuf2, acc.at[pl.ds(r0 + z * GM, GM)])

    mbase = t * M_PER_TILE
    msrc = lambda g: edge_hbm.at[pl.ds(mbase + g * GM, GM)]
    misrc = lambda g: midx_hbm.at[pl.ds(N_MESH + mbase + g * GM, GM)]
    pltpu.async_copy(msrc(0), ebuf0, fsem0)
    pltpu.async_copy(misrc(0), ibuf0, fsem0)
    pltpu.async_copy(msrc(1), ebuf1, fsem1)
    pltpu.async_copy(misrc(1), ibuf1, fsem1)
    zero_acc_slice()
    plsc.subcore_barrier()

    run_phase(GM_OUT, msrc, misrc, GM, primed=True)
    wbase = t * W_PER_TILE
    wsrc = lambda g: world_hbm.at[pl.ds(wbase + g * GW, GW)]
    wisrc = lambda g: widx_hbm.at[pl.ds(N_WORLD + wbase + g * GW, GW)]
    pltpu.async_copy(wsrc(0), ebuf0.at[pl.ds(0, GW)], fsem0)
    pltpu.async_copy(wisrc(0), wibuf0, fsem0)
    pltpu.async_copy(wsrc(1), ebuf1.at[pl.ds(0, GW)], fsem1)
    pltpu.async_copy(wisrc(1), wibuf1, fsem1)

    plsc.subcore_barrier()
    pltpu.sync_copy(acc.at[pl.ds(r0, ROWS_PER_TILE)],
                    mesh_out.at[c, pl.ds(r0, ROWS_PER_TILE)])
    plsc.subcore_barrier()

    zero_acc_slice()
    plsc.subcore_barrier()

    run_phase(GW_OUT, wsrc, wisrc, GW, primed=True)
    plsc.subcore_barrier()
    pltpu.sync_copy(acc.at[pl.ds(r0, ROWS_PER_TILE)],
                    world_out.at[c, pl.ds(r0, ROWS_PER_TILE)])

  return k(edge_attr, mesh_idx, world_attr, world_idx, zeros)


ROWS_BLK = 1000


def _tc_mlp_body(x, mp0, mp1, wp0, wp1, w1a, w1b, w1c, b1, w2, b2, out):
  m = mp0[0] + mp1[0]
  w = wp0[0] + wp1[0]
  h = (jnp.dot(x[...], w1a[...], preferred_element_type=jnp.float32)
       + jnp.dot(m, w1b[...], preferred_element_type=jnp.float32)
       + jnp.dot(w, w1c[...], preferred_element_type=jnp.float32)
       + b1[...])
  h = jnp.maximum(h, 0.0)
  out[...] = (jnp.dot(h, w2[...], preferred_element_type=jnp.float32)
              + b2[...])


def _tc_mlp(x, mesh_parts, world_parts, W1, b1, W2, b2):
  w1a, w1b, w1c = W1[:D], W1[D:2 * D], W1[2 * D:]
  b1r = b1.reshape(1, D)
  b2r = b2.reshape(1, D)
  rows_spec = pl.BlockSpec((ROWS_BLK, D), lambda i: (i, 0))
  part0_spec = pl.BlockSpec((1, ROWS_BLK, D), lambda i: (0, i, 0))
  part1_spec = pl.BlockSpec((1, ROWS_BLK, D), lambda i: (1, i, 0))
  full_spec = pl.BlockSpec((D, D), lambda i: (0, 0))
  bias_spec = pl.BlockSpec((1, D), lambda i: (0, 0))
  return pl.pallas_call(
      _tc_mlp_body,
      grid=(N_NODES // ROWS_BLK,),
      in_specs=[rows_spec, part0_spec, part1_spec, part0_spec, part1_spec,
                full_spec, full_spec, full_spec, bias_spec, full_spec,
                bias_spec],
      out_specs=rows_spec,
      out_shape=jax.ShapeDtypeStruct((N_NODES, D), jnp.float32),
  )(x, mesh_parts, mesh_parts, world_parts, world_parts,
    w1a, w1b, w1c, b1r, W2, b2r)


def kernel(x, edge_attr, edge_index, world_edge_attr, world_edge_index,
           W1, b1, W2, b2):
  mesh_idx = edge_index.astype(jnp.int32).reshape(-1)
  world_idx = world_edge_index.astype(jnp.int32).reshape(-1)
  zeros = jnp.zeros((GM, D), jnp.float32)
  mesh_parts, world_parts = _sc_aggregate(
      edge_attr, mesh_idx, world_edge_attr, world_idx, zeros)
  return _tc_mlp(x, mesh_parts, world_parts, W1, b1, W2, b2)


# speedup vs baseline: 1.2501x; 1.0315x over previous
import functools

import jax
import jax.numpy as jnp
from jax import lax
from jax.experimental import pallas as pl
from jax.experimental.pallas import tpu as pltpu
from jax.experimental.pallas import tpu_sc as plsc

N_NODES = 10000
N_MESH = 320000
N_WORLD = 32000
D = 128

NC = 2
NS = 16
NW = NC * NS

MB = 80
M_PER_TILE = N_MESH // NW
M_CHUNKS = M_PER_TILE // MB
WB = 40
W_PER_TILE = N_WORLD // NW
W_CHUNKS = W_PER_TILE // WB

GM = 80
GM_OUT = M_PER_TILE // GM
M_SUBS = GM // MB
GW = 40
GW_OUT = W_PER_TILE // GW
W_SUBS = GW // WB
N_PAD = 10240
ROWS_PER_TILE = N_PAD // NS


def _sc_aggregate(edge_attr, mesh_idx, world_attr, world_idx, zeros):
  mesh = plsc.VectorSubcoreMesh(core_axis_name="c", subcore_axis_name="s",
                                num_cores=NC, num_subcores=NS)

  @functools.partial(
      pl.kernel,
      out_type=[
          jax.ShapeDtypeStruct((NC, N_PAD, D), jnp.float32),
          jax.ShapeDtypeStruct((NC, N_PAD, D), jnp.float32),
      ],
      mesh=mesh,
      scratch_types=[
          pltpu.VMEM((GM, D), jnp.float32),
          pltpu.VMEM((GM, D), jnp.float32),
          pltpu.VMEM((GM, D), jnp.float32),
          pltpu.VMEM((MB,), jnp.int32),
          pltpu.VMEM((MB,), jnp.int32),
          pltpu.VMEM((MB,), jnp.int32),
          pltpu.VMEM((WB,), jnp.int32),
          pltpu.VMEM((WB,), jnp.int32),
          pltpu.VMEM((WB,), jnp.int32),
          pltpu.VMEM_SHARED((N_PAD, D), jnp.float32),
          pltpu.SemaphoreType.DMA,
          pltpu.SemaphoreType.DMA,
          pltpu.SemaphoreType.DMA,
          pltpu.SemaphoreType.DMA,
          pltpu.SemaphoreType.DMA,
          pltpu.SemaphoreType.DMA,
      ],
  )
  def k(edge_hbm, midx_hbm, world_hbm, widx_hbm, zeros_hbm,
        mesh_out, world_out, ebuf0, ebuf1, ebuf2, ibuf0, ibuf1, ibuf2,
        wibuf0, wibuf1, wibuf2, acc,
        fsem0, fsem1, fsem2, ssem0, ssem1, ssem2):
    c = lax.axis_index("c")
    s = lax.axis_index("s")
    t = c * NS + s
    r0 = s * ROWS_PER_TILE
    ebufs = (ebuf0, ebuf1, ebuf2)
    ibufs = (ibuf0, ibuf1, ibuf2)
    wibufs = (wibuf0, wibuf1, wibuf2)
    fsems = (fsem0, fsem1, fsem2)
    ssems = (ssem0, ssem1, ssem2)

    def run_phase(nout, src, isrc, rows, primed=False):
      def buf(b):
        return ebufs[b] if rows == GM else ebufs[b].at[pl.ds(0, rows)]

      def ibuf(b):
        return ibufs[b] if rows == GM else wibufs[b]

      def fire_fetch(g, b):
        pltpu.async_copy(src(g), buf(b), fsems[b])
        pltpu.async_copy(isrc(g), ibuf(b), fsems[b])

      def slot(u, bu, first=False, fire=True):
        pltpu.make_async_copy(src(u), buf(bu), fsems[bu]).wait()
        pltpu.make_async_copy(isrc(u), ibuf(bu), fsems[bu]).wait()
        pltpu.async_copy(buf(bu), acc.at[ibuf(bu)], ssems[bu], add=True)
        bp = (bu + 2) % 3
        if not first:
          pltpu.make_async_copy(buf(bp), acc.at[ibuf(bp)], ssems[bp]).wait()
        if fire:
          fire_fetch(u + 2, bp)

      if not primed:
        fire_fetch(0, 0)
        fire_fetch(1, 1)
      slot(0, 0, first=True)

      n_main = ((nout - 4 - 1) // 3) * 3

      @pl.loop(1, 1 + n_main, step=3)
      def _(g):
        slot(g, 1 % 3)
        slot(g + 1, 2 % 3)
        slot(g + 2, 0)

      for u in range(1 + n_main, nout):
        slot(u, u % 3, fire=(u + 2 < nout))
      b_last = (nout - 1) % 3
      pltpu.make_async_copy(buf(b_last), acc.at[ibuf(b_last)],
                            ssems[b_last]).wait()

    def zero_acc_slice():
      pltpu.sync_copy(zeros_hbm, ebuf2)
      for z in range(ROWS_PER_TILE // GM):
        pltpu.sync_copy(ebuf2, acc.at[pl.ds(r0 + z * GM, GM)])

    mbase = t * M_PER_TILE
    msrc = lambda g: edge_hbm.at[pl.ds(mbase + g * GM, GM)]
    misrc = lambda g: midx_hbm.at[pl.ds(N_MESH + mbase + g * GM, GM)]
    pltpu.async_copy(msrc(0), ebuf0, fsem0)
    pltpu.async_copy(misrc(0), ibuf0, fsem0)
    pltpu.async_copy(msrc(1), ebuf1, fsem1)
    pltpu.async_copy(misrc(1), ibuf1, fsem1)
    zero_acc_slice()
    plsc.subcore_barrier()

    run_phase(GM_OUT, msrc, misrc, GM, primed=True)
    wbase = t * W_PER_TILE
    wsrc = lambda g: world_hbm.at[pl.ds(wbase + g * GW, GW)]
    wisrc = lambda g: widx_hbm.at[pl.ds(N_WORLD + wbase + g * GW, GW)]
    pltpu.async_copy(wsrc(0), ebuf0.at[pl.ds(0, GW)], fsem0)
    pltpu.async_copy(wisrc(0), wibuf0, fsem0)
    pltpu.async_copy(wsrc(1), ebuf1.at[pl.ds(0, GW)], fsem1)
    pltpu.async_copy(wisrc(1), wibuf1, fsem1)

    plsc.subcore_barrier()
    pltpu.sync_copy(acc.at[pl.ds(r0, ROWS_PER_TILE)],
                    mesh_out.at[c, pl.ds(r0, ROWS_PER_TILE)])

    zero_acc_slice()
    plsc.subcore_barrier()

    run_phase(GW_OUT, wsrc, wisrc, GW, primed=True)
    plsc.subcore_barrier()
    pltpu.sync_copy(acc.at[pl.ds(r0, ROWS_PER_TILE)],
                    world_out.at[c, pl.ds(r0, ROWS_PER_TILE)])

  return k(edge_attr, mesh_idx, world_attr, world_idx, zeros)


ROWS_BLK = 2000


def _tc_mlp_body(x, mp0, mp1, wp0, wp1, w1a, w1b, w1c, b1, w2, b2, out):
  m = mp0[0] + mp1[0]
  w = wp0[0] + wp1[0]
  h = (jnp.dot(x[...], w1a[...], preferred_element_type=jnp.float32)
       + jnp.dot(m, w1b[...], preferred_element_type=jnp.float32)
       + jnp.dot(w, w1c[...], preferred_element_type=jnp.float32)
       + b1[...])
  h = jnp.maximum(h, 0.0)
  out[...] = (jnp.dot(h, w2[...], preferred_element_type=jnp.float32)
              + b2[...])


def _tc_mlp(x, mesh_parts, world_parts, W1, b1, W2, b2):
  w1a, w1b, w1c = W1[:D], W1[D:2 * D], W1[2 * D:]
  b1r = b1.reshape(1, D)
  b2r = b2.reshape(1, D)
  rows_spec = pl.BlockSpec((ROWS_BLK, D), lambda i: (i, 0))
  part0_spec = pl.BlockSpec((1, ROWS_BLK, D), lambda i: (0, i, 0))
  part1_spec = pl.BlockSpec((1, ROWS_BLK, D), lambda i: (1, i, 0))
  full_spec = pl.BlockSpec((D, D), lambda i: (0, 0))
  bias_spec = pl.BlockSpec((1, D), lambda i: (0, 0))
  return pl.pallas_call(
      _tc_mlp_body,
      grid=(N_NODES // ROWS_BLK,),
      in_specs=[rows_spec, part0_spec, part1_spec, part0_spec, part1_spec,
                full_spec, full_spec, full_spec, bias_spec, full_spec,
                bias_spec],
      out_specs=rows_spec,
      out_shape=jax.ShapeDtypeStruct((N_NODES, D), jnp.float32),
  )(x, mesh_parts, mesh_parts, world_parts, world_parts,
    w1a, w1b, w1c, b1r, W2, b2r)


def kernel(x, edge_attr, edge_index, world_edge_attr, world_edge_index,
           W1, b1, W2, b2):
  mesh_idx = edge_index.astype(jnp.int32).reshape(-1)
  world_idx = world_edge_index.astype(jnp.int32).reshape(-1)
  zeros = jnp.zeros((GM, D), jnp.float32)
  mesh_parts, world_parts = _sc_aggregate(
      edge_attr, mesh_idx, world_edge_attr, world_idx, zeros)
  return _tc_mlp(x, mesh_parts, world_parts, W1, b1, W2, b2)
